# baseline (speedup 1.0000x reference)
import functools

import jax
import jax.numpy as jnp
from jax import lax
from jax.experimental import pallas as pl
from jax.experimental.pallas import tpu as pltpu
from jax.experimental.pallas import tpu_sc as plsc

N_CHROM = 4
CHROM = 25000
D_IN = 256
DIM = 128
EPS = 1e-5

_NC = 2
_NS = 16
_NW = _NC * _NS
_LANES = 16

_GCHUNK = 128


def _sc_gather(table_flat, x):
    b = x.shape[0]
    b_per_w = b // _NW
    n_chunks = b_per_w // _GCHUNK
    d = table_flat.shape[1]
    mesh = plsc.VectorSubcoreMesh(core_axis_name="c", subcore_axis_name="s")

    @functools.partial(
        pl.kernel,
        mesh=mesh,
        out_type=jax.ShapeDtypeStruct((b, d), jnp.float32),
        scratch_types=[
            pltpu.VMEM((b_per_w,), jnp.int32),
            pltpu.VMEM((n_chunks, _GCHUNK), jnp.int32),
            pltpu.VMEM((_GCHUNK, d), jnp.float32),
            pltpu.VMEM((_GCHUNK, d), jnp.float32),
            pltpu.SemaphoreType.DMA,
            pltpu.SemaphoreType.DMA,
        ],
    )
    def k(table_hbm, x_hbm, out_hbm, x_v, idx_v, rows0_v, rows1_v, sem0, sem1):
        wid = lax.axis_index("s") * _NC + lax.axis_index("c")
        base = wid * b_per_w
        pltpu.sync_copy(x_hbm.at[pl.ds(base, b_per_w)], x_v)
        for j in range(n_chunks):
            for i in range(_GCHUNK // _LANES):
                v = x_v[pl.ds(j * _GCHUNK + i * _LANES, _LANES)]
                idx_v[j, pl.ds(i * _LANES, _LANES)] = jnp.maximum(
                    v - 1, jnp.zeros_like(v)
                )
        rows = (rows0_v, rows1_v)
        sems = (sem0, sem1)
        copies = [None, None]
        for j in range(n_chunks):
            s = j % 2
            if copies[s] is not None:
                copies[s].wait()
                pltpu.sync_copy(
                    rows[s], out_hbm.at[pl.ds(base + (j - 2) * _GCHUNK, _GCHUNK)]
                )
            cp = pltpu.async_copy(table_hbm.at[idx_v.at[j]], rows[s], sems[s])
            copies[s] = cp
        for j in range(n_chunks - 2, n_chunks):
            s = j % 2
            copies[s].wait()
            pltpu.sync_copy(
                rows[s], out_hbm.at[pl.ds(base + j * _GCHUNK, _GCHUNK)]
            )

    return k(table_flat, x)


def _tc_body(x_ref, xr_ref, g_ref, w_ref, nw_ref, nb_ref, out_ref,
             enc_scr, stat_scr, *, bblk):
    p = pl.program_id(0)
    b = pl.program_id(1)

    @pl.when((p == 0) & (b == 0))
    def _init():
        stat_scr[...] = jnp.zeros_like(stat_scr)

    xb = x_ref[...]
    bounds = [(i * CHROM + 1, (i + 1) * CHROM) for i in range(N_CHROM)]
    col_masks = [(xb >= lo) & (xb <= hi) for lo, hi in bounds]

    @pl.when(p == 0)
    def _pass1():
        xr = xr_ref[...]
        mf_t = jnp.concatenate(
            [((xr >= lo) & (xr <= hi)).astype(jnp.bfloat16)
             for lo, hi in bounds], axis=0)
        gb = g_ref[...].astype(jnp.bfloat16)
        e_all = jnp.dot(gb, w_ref[...],
                        preferred_element_type=jnp.float32
                        ).astype(jnp.bfloat16)
        esel = jnp.zeros((bblk, DIM), jnp.bfloat16)
        for i in range(N_CHROM):
            esel = jnp.where(col_masks[i], e_all[:, i * DIM:(i + 1) * DIM],
                             esel)
        stat_scr[:, 0:DIM] += jnp.dot(mf_t, esel,
                                      preferred_element_type=jnp.float32)
        stat_scr[:, DIM:2 * DIM] += jnp.dot(
            mf_t, esel * esel, preferred_element_type=jnp.float32)
        for i in range(N_CHROM):
            ci = jnp.sum(col_masks[i].astype(jnp.float32))
            stat_scr[pl.ds(i, 1), pl.ds(2 * DIM, DIM)] = (
                stat_scr[pl.ds(i, 1), pl.ds(2 * DIM, DIM)] + ci)
        enc_scr[pl.ds(b * bblk, bblk), :] = esel

    @pl.when(p == 1)
    def _pass2():
        stats = stat_scr[...]
        cnt = jnp.maximum(stats[:, 2 * DIM:3 * DIM], 1.0)
        mean = stats[:, 0:DIM] / cnt
        var = stats[:, DIM:2 * DIM] / cnt - mean * mean
        rstd = lax.rsqrt(var + EPS)
        rs = jnp.concatenate([rstd, mean * rstd], axis=1)
        mf = jnp.concatenate(
            [m.astype(jnp.float32) for m in col_masks], axis=1)
        c = jnp.dot(mf, rs, preferred_element_type=jnp.float32)
        enc = enc_scr[pl.ds(b * bblk, bblk), :].astype(jnp.float32)
        normalized = enc * c[:, 0:DIM] - c[:, DIM:2 * DIM]
        t = jnp.tanh(normalized)
        out_ref[...] = jnp.dot(t, nw_ref[...],
                               preferred_element_type=jnp.float32) + nb_ref[...]


def _tc_forward(g, x2, xr, w_cat, next_wt, next_b2):
    b = g.shape[0]
    bblk = 1024
    nb = b // bblk
    grid = (2, nb)
    return pl.pallas_call(
        functools.partial(_tc_body, bblk=bblk),
        grid=grid,
        in_specs=[
            pl.BlockSpec((bblk, 1), lambda p, i: (i, 0)),
            pl.BlockSpec((1, bblk), lambda p, i: (0, i)),
            pl.BlockSpec((bblk, D_IN), lambda p, i: (i * (1 - p), 0)),
            pl.BlockSpec((D_IN, N_CHROM * DIM), lambda p, i: (0, 0)),
            pl.BlockSpec((DIM, DIM), lambda p, i: (0, 0)),
            pl.BlockSpec((1, DIM), lambda p, i: (0, 0)),
        ],
        out_specs=pl.BlockSpec((bblk, DIM), lambda p, i: (i, 0)),
        out_shape=jax.ShapeDtypeStruct((b, DIM), jnp.float32),
        scratch_shapes=[
            pltpu.VMEM((b, DIM), jnp.bfloat16),
            pltpu.VMEM((N_CHROM, 3 * DIM), jnp.float32),
        ],
    )(x2, xr, g, w_cat, next_wt, next_b2)


def kernel(x, tables, Ws, next_W, next_b):
    b = x.shape[0]
    table_flat = tables.reshape(N_CHROM * CHROM, D_IN)
    g = _sc_gather(table_flat, x)
    x2 = x.reshape(b, 1)
    xr = x.reshape(1, b)
    w_cat = jnp.transpose(Ws, (2, 0, 1)).reshape(
        D_IN, N_CHROM * DIM).astype(jnp.bfloat16)
    next_wt = next_W.T
    next_b2 = next_b.reshape(1, DIM)
    return _tc_forward(g, x2, xr, w_cat, next_wt, next_b2)


# speedup vs baseline: 1.0789x; 1.0789x over previous
import functools

import jax
import jax.numpy as jnp
from jax import lax
from jax.experimental import pallas as pl
from jax.experimental.pallas import tpu as pltpu
from jax.experimental.pallas import tpu_sc as plsc

N_CHROM = 4
CHROM = 25000
D_IN = 256
DIM = 128
EPS = 1e-5

_NC = 2
_NS = 16
_NW = _NC * _NS
_LANES = 16

_GCHUNK = 128


def _sc_gather(table_flat, x):
    b = x.shape[0]
    b_per_w = b // _NW
    n_chunks = b_per_w // _GCHUNK
    d = table_flat.shape[1]
    mesh = plsc.VectorSubcoreMesh(core_axis_name="c", subcore_axis_name="s")

    @functools.partial(
        pl.kernel,
        mesh=mesh,
        out_type=jax.ShapeDtypeStruct((b, d), jnp.float32),
        scratch_types=[
            pltpu.VMEM((b_per_w,), jnp.int32),
            pltpu.VMEM((n_chunks, _GCHUNK), jnp.int32),
            pltpu.VMEM((_GCHUNK, d), jnp.float32),
            pltpu.VMEM((_GCHUNK, d), jnp.float32),
            pltpu.SemaphoreType.DMA,
            pltpu.SemaphoreType.DMA,
        ],
    )
    def k(table_hbm, x_hbm, out_hbm, x_v, idx_v, rows0_v, rows1_v, sem0, sem1):
        wid = lax.axis_index("s") * _NC + lax.axis_index("c")
        base = wid * b_per_w
        pltpu.sync_copy(x_hbm.at[pl.ds(base, b_per_w)], x_v)
        for j in range(n_chunks):
            for i in range(_GCHUNK // _LANES):
                v = x_v[pl.ds(j * _GCHUNK + i * _LANES, _LANES)]
                idx_v[j, pl.ds(i * _LANES, _LANES)] = jnp.maximum(
                    v - 1, jnp.zeros_like(v)
                )
        rows = (rows0_v, rows1_v)
        sems = (sem0, sem1)
        copies = [None, None]
        for j in range(n_chunks):
            s = j % 2
            if copies[s] is not None:
                copies[s].wait()
                pltpu.sync_copy(
                    rows[s], out_hbm.at[pl.ds(base + (j - 2) * _GCHUNK, _GCHUNK)]
                )
            cp = pltpu.async_copy(table_hbm.at[idx_v.at[j]], rows[s], sems[s])
            copies[s] = cp
        for j in range(n_chunks - 2, n_chunks):
            s = j % 2
            copies[s].wait()
            pltpu.sync_copy(
                rows[s], out_hbm.at[pl.ds(base + j * _GCHUNK, _GCHUNK)]
            )

    return k(table_flat, x)


def _tc_body(x_ref, xr_ref, g_ref, w_ref, nw_ref, nb_ref, out_ref,
             enc_scr, stat_scr, *, bblk):
    p = pl.program_id(0)
    b = pl.program_id(1)

    @pl.when((p == 0) & (b == 0))
    def _init():
        stat_scr[...] = jnp.zeros_like(stat_scr)

    xb = x_ref[...]
    m01 = xb <= 2 * CHROM
    m0 = xb <= CHROM
    m2 = xb <= 3 * CHROM

    @pl.when(p == 0)
    def _pass1():
        xr = xr_ref[...]
        bounds = [(i * CHROM + 1, (i + 1) * CHROM) for i in range(N_CHROM)]
        row_masks = [(xr >= lo) & (xr <= hi) for lo, hi in bounds]
        mf_t = jnp.concatenate(
            [m.astype(jnp.float32) for m in row_masks], axis=0)
        gb = g_ref[...].astype(jnp.bfloat16)
        e_all = jnp.dot(gb, w_ref[...],
                        preferred_element_type=jnp.float32)
        e0 = e_all[:, 0:DIM]
        e1 = e_all[:, DIM:2 * DIM]
        e2 = e_all[:, 2 * DIM:3 * DIM]
        e3 = e_all[:, 3 * DIM:4 * DIM]
        esel = jnp.where(m01, jnp.where(m0, e0, e1),
                         jnp.where(m2, e2, e3))
        stat_scr[:, 0:DIM] += jnp.dot(mf_t, esel,
                                      preferred_element_type=jnp.float32)
        stat_scr[:, DIM:2 * DIM] += jnp.dot(
            mf_t, esel * esel, preferred_element_type=jnp.float32)
        for i in range(N_CHROM):
            ci = jnp.sum(mf_t[i:i + 1, :])
            stat_scr[pl.ds(i, 1), pl.ds(2 * DIM, DIM)] = (
                stat_scr[pl.ds(i, 1), pl.ds(2 * DIM, DIM)] + ci)
        enc_scr[pl.ds(b * bblk, bblk), :] = esel

    @pl.when(p == 1)
    def _pass2():
        stats = stat_scr[...]
        cnt = jnp.maximum(stats[:, 2 * DIM:3 * DIM], 1.0)
        mean = stats[:, 0:DIM] / cnt
        var = stats[:, DIM:2 * DIM] / cnt - mean * mean
        rstd = lax.rsqrt(var + EPS)
        shift = mean * rstd
        valid = xb >= 1
        scale = jnp.where(
            m01,
            jnp.where(m0, rstd[0:1, :], rstd[1:2, :]),
            jnp.where(m2, rstd[2:3, :], rstd[3:4, :]))
        offs = jnp.where(
            m01,
            jnp.where(m0, shift[0:1, :], shift[1:2, :]),
            jnp.where(m2, shift[2:3, :], shift[3:4, :]))
        scale = jnp.where(valid, scale, 0.0)
        offs = jnp.where(valid, offs, 0.0)
        enc = enc_scr[pl.ds(b * bblk, bblk), :]
        normalized = enc * scale - offs
        t = jnp.tanh(normalized)
        out_ref[...] = jnp.dot(t, nw_ref[...],
                               preferred_element_type=jnp.float32) + nb_ref[...]


def _tc_forward(g, x2, xr, w_cat, next_wt, next_b2):
    b = g.shape[0]
    bblk = 1024
    nb = b // bblk
    grid = (2, nb)
    return pl.pallas_call(
        functools.partial(_tc_body, bblk=bblk),
        grid=grid,
        in_specs=[
            pl.BlockSpec((bblk, 1), lambda p, i: (i, 0)),
            pl.BlockSpec((1, bblk), lambda p, i: (0, i)),
            pl.BlockSpec((bblk, D_IN), lambda p, i: (i * (1 - p), 0)),
            pl.BlockSpec((D_IN, N_CHROM * DIM), lambda p, i: (0, 0)),
            pl.BlockSpec((DIM, DIM), lambda p, i: (0, 0)),
            pl.BlockSpec((1, DIM), lambda p, i: (0, 0)),
        ],
        out_specs=pl.BlockSpec((bblk, DIM), lambda p, i: (i, 0)),
        out_shape=jax.ShapeDtypeStruct((b, DIM), jnp.float32),
        scratch_shapes=[
            pltpu.VMEM((b, DIM), jnp.float32),
            pltpu.VMEM((N_CHROM, 3 * DIM), jnp.float32),
        ],
    )(x2, xr, g, w_cat, next_wt, next_b2)


def kernel(x, tables, Ws, next_W, next_b):
    b = x.shape[0]
    table_flat = tables.reshape(N_CHROM * CHROM, D_IN)
    g = _sc_gather(table_flat, x)
    x2 = x.reshape(b, 1)
    xr = x.reshape(1, b)
    w_cat = jnp.transpose(Ws, (2, 0, 1)).reshape(
        D_IN, N_CHROM * DIM).astype(jnp.bfloat16)
    next_wt = next_W.T
    next_b2 = next_b.reshape(1, DIM)
    return _tc_forward(g, x2, xr, w_cat, next_wt, next_b2)
